# baseline (speedup 1.0000x reference)
import functools

import jax
import jax.numpy as jnp
from jax import lax
from jax.experimental import pallas as pl
from jax.experimental.pallas import tpu as pltpu
from jax.experimental.pallas import tpu_sc as plsc

_DIM = 64
_L = 16
_MARGIN = 1.0


_BLK = 2048
_HW = _BLK // 2


def _pack_body(x, out):
    eye = jnp.eye(_DIM, dtype=jnp.float32)
    t = lax.dot_general(x[...], eye, (((0,), (0,)), ((), ())),
                        preferred_element_type=jnp.float32)
    out[:, :_DIM] = t[:_HW]
    out[:, _DIM:] = t[_HW:]


def _pack_table(tab_t):
    n = tab_t.shape[1]
    grid = (n + _BLK - 1) // _BLK
    return pl.pallas_call(
        _pack_body,
        grid=(grid,),
        in_specs=[pl.BlockSpec((_DIM, _BLK), lambda j: (0, j))],
        out_specs=pl.BlockSpec((_HW, 2 * _DIM), lambda j: (j, 0)),
        out_shape=jax.ShapeDtypeStruct((grid * _HW, 2 * _DIM), jnp.float32),
    )(tab_t)


def _transe_body(C, n_chunks,
                 rph, rpr, rpt, rnh, rnr, rnt, entp, relp, out,
                 iph, ipr, ipt, inh, inr, int_,
                 sph, spr, spt, snh, snr, snt,
                 vph, vpr, vpt, vnh, vnr, vnt, sbuf, acc_ref, sem):
    nc = 2
    wid = lax.axis_index("s") * nc + lax.axis_index("c")
    base = wid * (C * n_chunks)
    lane = lax.iota(jnp.int32, _L)

    acc_ref[...] = jnp.zeros((_L,), jnp.float32)
    for c in range(n_chunks):
        off = base + c * C
        pltpu.sync_copy(rph.at[pl.ds(off, C)], iph)
        pltpu.sync_copy(rpr.at[pl.ds(off, C)], ipr)
        pltpu.sync_copy(rpt.at[pl.ds(off, C)], ipt)
        pltpu.sync_copy(rnh.at[pl.ds(off, C)], inh)
        pltpu.sync_copy(rnr.at[pl.ds(off, C)], inr)
        pltpu.sync_copy(rnt.at[pl.ds(off, C)], int_)
        for src, dst in ((iph, sph), (ipr, spr), (ipt, spt),
                         (inh, snh), (inr, snr), (int_, snt)):
            for s in range(C // _L):
                sl = pl.ds(s * _L, _L)
                iv = src[sl]
                dst[sl] = (lax.shift_right_logical(iv, 11) * _HW
                           + (iv & (_HW - 1)))
        d0 = pltpu.async_copy(entp.at[sph], vph, sem)
        d1 = pltpu.async_copy(relp.at[spr], vpr, sem)
        d2 = pltpu.async_copy(entp.at[spt], vpt, sem)
        d3 = pltpu.async_copy(entp.at[snh], vnh, sem)
        d4 = pltpu.async_copy(relp.at[snr], vnr, sem)
        d5 = pltpu.async_copy(entp.at[snt], vnt, sem)
        d0.wait(); d1.wait(); d2.wait(); d3.wait(); d4.wait(); d5.wait()

        def group(g, carry):
            gs = pl.ds(g * _L, _L)
            hph, hpr, hpt = iph[gs] & _HW, ipr[gs] & _HW, ipt[gs] & _HW
            hnh, hnr, hnt = inh[gs] & _HW, inr[gs] & _HW, int_[gs] & _HW
            for j in range(_L):
                i = g * _L + j
                s = None
                for k in range(_DIM // _L):
                    sl = pl.ds(k * _L, _L)
                    sh = pl.ds(_DIM + k * _L, _L)
                    ph = jnp.where(hph[j] != 0, vph[i, sh], vph[i, sl])
                    pr = jnp.where(hpr[j] != 0, vpr[i, sh], vpr[i, sl])
                    pt = jnp.where(hpt[j] != 0, vpt[i, sh], vpt[i, sl])
                    nh = jnp.where(hnh[j] != 0, vnh[i, sh], vnh[i, sl])
                    nr = jnp.where(hnr[j] != 0, vnr[i, sh], vnr[i, sl])
                    nt = jnp.where(hnt[j] != 0, vnt[i, sh], vnt[i, sl])
                    d = jnp.abs(ph + pr - pt) - jnp.abs(nh + nr - nt)
                    s = d if s is None else s + d
                plsc.store_scatter(sbuf, [lane * _L + j], s)
            tot = None
            for k in range(_L):
                v = sbuf[pl.ds(k * _L, _L)]
                tot = v if tot is None else tot + v
            hinge = jnp.maximum(tot + _MARGIN, 0.0)
            acc_ref[...] = acc_ref[...] + hinge
            return carry

        lax.fori_loop(0, C // _L, group, jnp.int32(0))

    pltpu.sync_copy(acc_ref, out.at[pl.ds(wid * _L, _L)])


def _transe_sc(rph, rpr, rpt, rnh, rnr, rnt, entp, relp):
    B = rph.shape[0]
    nw = 32
    C = 64
    n_chunks = B // (nw * C)
    mesh = plsc.VectorSubcoreMesh(core_axis_name="c", subcore_axis_name="s")
    idx_t = pltpu.VMEM((C,), jnp.int32)
    row_t = pltpu.VMEM((C, 2 * _DIM), jnp.float32)
    kern = pl.kernel(
        functools.partial(_transe_body, C, n_chunks),
        mesh=mesh,
        compiler_params=pltpu.CompilerParams(needs_layout_passes=False),
        out_type=jax.ShapeDtypeStruct((nw * _L,), jnp.float32),
        scratch_types=[idx_t] * 12 + [row_t] * 6 + [
            pltpu.VMEM((_L * _L,), jnp.float32),
            pltpu.VMEM((_L,), jnp.float32),
            pltpu.SemaphoreType.DMA,
        ],
    )
    return kern(rph, rpr, rpt, rnh, rnr, rnt, entp, relp)


def kernel(r_p_h, r_p_r, r_p_t, r_n_h, r_n_r, r_n_t, ent_embed, rel_embed):
    B = r_p_h.shape[0]
    entp = _pack_table(ent_embed.T)
    relp = _pack_table(rel_embed.T)
    partials = _transe_sc(
        r_p_h.astype(jnp.int32), r_p_r.astype(jnp.int32),
        r_p_t.astype(jnp.int32), r_n_h.astype(jnp.int32),
        r_n_r.astype(jnp.int32), r_n_t.astype(jnp.int32),
        entp, relp)
    return jnp.sum(partials) * jnp.float32(1.0 / B)


# speedup vs baseline: 1.5882x; 1.5882x over previous
import functools

import jax
import jax.numpy as jnp
from jax import lax
from jax.experimental import pallas as pl
from jax.experimental.pallas import tpu as pltpu
from jax.experimental.pallas import tpu_sc as plsc

_DIM = 64
_L = 16
_MARGIN = 1.0


_BLK = 8192
_SH = 13
_HW = _BLK // 2


def _pack_body(x, out):
    out[:, :_DIM] = x[:, :_HW].T
    out[:, _DIM:] = x[:, _HW:].T


def _pack_table(tab_t):
    n = tab_t.shape[1]
    grid = (n + _BLK - 1) // _BLK
    return pl.pallas_call(
        _pack_body,
        grid=(grid,),
        in_specs=[pl.BlockSpec((_DIM, _BLK), lambda j: (0, j))],
        out_specs=pl.BlockSpec((_HW, 2 * _DIM), lambda j: (j, 0)),
        out_shape=jax.ShapeDtypeStruct((grid * _HW, 2 * _DIM), jnp.float32),
    )(tab_t)


def _transe_body(C, n_chunks,
                 rph, rpr, rpt, rnh, rnr, rnt, entp, relp, out,
                 iph, ipr, ipt, inh, inr, int_,
                 sph, spr, spt, snh, snr, snt,
                 vph, vpr, vpt, vnh, vnr, vnt, sbuf, acc_ref, sem):
    nc = 2
    wid = lax.axis_index("s") * nc + lax.axis_index("c")
    base = wid * (C * n_chunks)
    lane = lax.iota(jnp.int32, _L)

    acc_ref[...] = jnp.zeros((_L,), jnp.float32)
    for c in range(n_chunks):
        off = base + c * C
        pltpu.sync_copy(rph.at[pl.ds(off, C)], iph)
        pltpu.sync_copy(rpr.at[pl.ds(off, C)], ipr)
        pltpu.sync_copy(rpt.at[pl.ds(off, C)], ipt)
        pltpu.sync_copy(rnh.at[pl.ds(off, C)], inh)
        pltpu.sync_copy(rnr.at[pl.ds(off, C)], inr)
        pltpu.sync_copy(rnt.at[pl.ds(off, C)], int_)
        for src, dst in ((iph, sph), (ipr, spr), (ipt, spt),
                         (inh, snh), (inr, snr), (int_, snt)):
            for s in range(C // _L):
                sl = pl.ds(s * _L, _L)
                iv = src[sl]
                dst[sl] = (lax.shift_right_logical(iv, _SH) * _HW
                           + (iv & (_HW - 1)))
        d0 = pltpu.async_copy(entp.at[sph], vph, sem)
        d1 = pltpu.async_copy(relp.at[spr], vpr, sem)
        d2 = pltpu.async_copy(entp.at[spt], vpt, sem)
        d3 = pltpu.async_copy(entp.at[snh], vnh, sem)
        d4 = pltpu.async_copy(relp.at[snr], vnr, sem)
        d5 = pltpu.async_copy(entp.at[snt], vnt, sem)
        d0.wait(); d1.wait(); d2.wait(); d3.wait(); d4.wait(); d5.wait()

        def group(g, carry):
            gs = pl.ds(g * _L, _L)
            hph, hpr, hpt = iph[gs] & _HW, ipr[gs] & _HW, ipt[gs] & _HW
            hnh, hnr, hnt = inh[gs] & _HW, inr[gs] & _HW, int_[gs] & _HW
            for j in range(_L):
                i = g * _L + j
                s = None
                for k in range(_DIM // _L):
                    sl = pl.ds(k * _L, _L)
                    sh = pl.ds(_DIM + k * _L, _L)
                    ph = jnp.where(hph[j] != 0, vph[i, sh], vph[i, sl])
                    pr = jnp.where(hpr[j] != 0, vpr[i, sh], vpr[i, sl])
                    pt = jnp.where(hpt[j] != 0, vpt[i, sh], vpt[i, sl])
                    nh = jnp.where(hnh[j] != 0, vnh[i, sh], vnh[i, sl])
                    nr = jnp.where(hnr[j] != 0, vnr[i, sh], vnr[i, sl])
                    nt = jnp.where(hnt[j] != 0, vnt[i, sh], vnt[i, sl])
                    d = jnp.abs(ph + pr - pt) - jnp.abs(nh + nr - nt)
                    s = d if s is None else s + d
                plsc.store_scatter(sbuf, [lane * _L + j], s)
            tot = None
            for k in range(_L):
                v = sbuf[pl.ds(k * _L, _L)]
                tot = v if tot is None else tot + v
            hinge = jnp.maximum(tot + _MARGIN, 0.0)
            acc_ref[...] = acc_ref[...] + hinge
            return carry

        lax.fori_loop(0, C // _L, group, jnp.int32(0))

    pltpu.sync_copy(acc_ref, out.at[pl.ds(wid * _L, _L)])


def _transe_sc(rph, rpr, rpt, rnh, rnr, rnt, entp, relp):
    B = rph.shape[0]
    nw = 32
    C = 64
    n_chunks = B // (nw * C)
    mesh = plsc.VectorSubcoreMesh(core_axis_name="c", subcore_axis_name="s")
    idx_t = pltpu.VMEM((C,), jnp.int32)
    row_t = pltpu.VMEM((C, 2 * _DIM), jnp.float32)
    kern = pl.kernel(
        functools.partial(_transe_body, C, n_chunks),
        mesh=mesh,
        compiler_params=pltpu.CompilerParams(needs_layout_passes=False),
        out_type=jax.ShapeDtypeStruct((nw * _L,), jnp.float32),
        scratch_types=[idx_t] * 12 + [row_t] * 6 + [
            pltpu.VMEM((_L * _L,), jnp.float32),
            pltpu.VMEM((_L,), jnp.float32),
            pltpu.SemaphoreType.DMA,
        ],
    )
    return kern(rph, rpr, rpt, rnh, rnr, rnt, entp, relp)


def kernel(r_p_h, r_p_r, r_p_t, r_n_h, r_n_r, r_n_t, ent_embed, rel_embed):
    B = r_p_h.shape[0]
    entp = _pack_table(ent_embed.T)
    relp = _pack_table(rel_embed.T)
    partials = _transe_sc(
        r_p_h.astype(jnp.int32), r_p_r.astype(jnp.int32),
        r_p_t.astype(jnp.int32), r_n_h.astype(jnp.int32),
        r_n_r.astype(jnp.int32), r_n_t.astype(jnp.int32),
        entp, relp)
    return jnp.sum(partials) * jnp.float32(1.0 / B)


# speedup vs baseline: 1.7946x; 1.1300x over previous
import functools

import jax
import jax.numpy as jnp
from jax import lax
from jax.experimental import pallas as pl
from jax.experimental.pallas import tpu as pltpu
from jax.experimental.pallas import tpu_sc as plsc

_DIM = 64
_L = 16
_MARGIN = 1.0


_BLK = 32768
_SH = 15
_HW = _BLK // 2


def _pack_body(x, out):
    out[:, :_DIM] = x[:, :_HW].T
    out[:, _DIM:] = x[:, _HW:].T


def _pack_table(tab_t):
    n = tab_t.shape[1]
    grid = (n + _BLK - 1) // _BLK
    return pl.pallas_call(
        _pack_body,
        grid=(grid,),
        in_specs=[pl.BlockSpec((_DIM, _BLK), lambda j: (0, j))],
        out_specs=pl.BlockSpec((_HW, 2 * _DIM), lambda j: (j, 0)),
        out_shape=jax.ShapeDtypeStruct((grid * _HW, 2 * _DIM), jnp.float32),
    )(tab_t)


def _transe_body(C, n_chunks,
                 rph, rpr, rpt, rnh, rnr, rnt, entp, relp, out,
                 iph, ipr, ipt, inh, inr, int_,
                 sph, spr, spt, snh, snr, snt,
                 vph, vpr, vpt, vnh, vnr, vnt, sbuf, acc_ref, sem):
    nc = 2
    wid = lax.axis_index("s") * nc + lax.axis_index("c")
    base = wid * (C * n_chunks)
    lane = lax.iota(jnp.int32, _L)

    acc_ref[...] = jnp.zeros((_L,), jnp.float32)
    for c in range(n_chunks):
        off = base + c * C
        pltpu.sync_copy(rph.at[pl.ds(off, C)], iph)
        pltpu.sync_copy(rpr.at[pl.ds(off, C)], ipr)
        pltpu.sync_copy(rpt.at[pl.ds(off, C)], ipt)
        pltpu.sync_copy(rnh.at[pl.ds(off, C)], inh)
        pltpu.sync_copy(rnr.at[pl.ds(off, C)], inr)
        pltpu.sync_copy(rnt.at[pl.ds(off, C)], int_)
        for src, dst in ((iph, sph), (ipr, spr), (ipt, spt),
                         (inh, snh), (inr, snr), (int_, snt)):
            for s in range(C // _L):
                sl = pl.ds(s * _L, _L)
                iv = src[sl]
                dst[sl] = (lax.shift_right_logical(iv, _SH) * _HW
                           + (iv & (_HW - 1)))
        d0 = pltpu.async_copy(entp.at[sph], vph, sem)
        d1 = pltpu.async_copy(relp.at[spr], vpr, sem)
        d2 = pltpu.async_copy(entp.at[spt], vpt, sem)
        d3 = pltpu.async_copy(entp.at[snh], vnh, sem)
        d4 = pltpu.async_copy(relp.at[snr], vnr, sem)
        d5 = pltpu.async_copy(entp.at[snt], vnt, sem)
        d0.wait(); d1.wait(); d2.wait(); d3.wait(); d4.wait(); d5.wait()

        def group(g, carry):
            gs = pl.ds(g * _L, _L)
            hph, hpr, hpt = iph[gs] & _HW, ipr[gs] & _HW, ipt[gs] & _HW
            hnh, hnr, hnt = inh[gs] & _HW, inr[gs] & _HW, int_[gs] & _HW
            for j in range(_L):
                i = g * _L + j
                s = None
                for k in range(_DIM // _L):
                    sl = pl.ds(k * _L, _L)
                    sh = pl.ds(_DIM + k * _L, _L)
                    ph = jnp.where(hph[j] != 0, vph[i, sh], vph[i, sl])
                    pr = jnp.where(hpr[j] != 0, vpr[i, sh], vpr[i, sl])
                    pt = jnp.where(hpt[j] != 0, vpt[i, sh], vpt[i, sl])
                    nh = jnp.where(hnh[j] != 0, vnh[i, sh], vnh[i, sl])
                    nr = jnp.where(hnr[j] != 0, vnr[i, sh], vnr[i, sl])
                    nt = jnp.where(hnt[j] != 0, vnt[i, sh], vnt[i, sl])
                    d = jnp.abs(ph + pr - pt) - jnp.abs(nh + nr - nt)
                    s = d if s is None else s + d
                plsc.store_scatter(sbuf, [lane * _L + j], s)
            tot = None
            for k in range(_L):
                v = sbuf[pl.ds(k * _L, _L)]
                tot = v if tot is None else tot + v
            hinge = jnp.maximum(tot + _MARGIN, 0.0)
            acc_ref[...] = acc_ref[...] + hinge
            return carry

        lax.fori_loop(0, C // _L, group, jnp.int32(0))

    pltpu.sync_copy(acc_ref, out.at[pl.ds(wid * _L, _L)])


def _transe_sc(rph, rpr, rpt, rnh, rnr, rnt, entp, relp):
    B = rph.shape[0]
    nw = 32
    C = 64
    n_chunks = B // (nw * C)
    mesh = plsc.VectorSubcoreMesh(core_axis_name="c", subcore_axis_name="s")
    idx_t = pltpu.VMEM((C,), jnp.int32)
    row_t = pltpu.VMEM((C, 2 * _DIM), jnp.float32)
    kern = pl.kernel(
        functools.partial(_transe_body, C, n_chunks),
        mesh=mesh,
        compiler_params=pltpu.CompilerParams(needs_layout_passes=False),
        out_type=jax.ShapeDtypeStruct((nw * _L,), jnp.float32),
        scratch_types=[idx_t] * 12 + [row_t] * 6 + [
            pltpu.VMEM((_L * _L,), jnp.float32),
            pltpu.VMEM((_L,), jnp.float32),
            pltpu.SemaphoreType.DMA,
        ],
    )
    return kern(rph, rpr, rpt, rnh, rnr, rnt, entp, relp)


def kernel(r_p_h, r_p_r, r_p_t, r_n_h, r_n_r, r_n_t, ent_embed, rel_embed):
    B = r_p_h.shape[0]
    entp = _pack_table(ent_embed.T)
    relp = _pack_table(rel_embed.T)
    partials = _transe_sc(
        r_p_h.astype(jnp.int32), r_p_r.astype(jnp.int32),
        r_p_t.astype(jnp.int32), r_n_h.astype(jnp.int32),
        r_n_r.astype(jnp.int32), r_n_t.astype(jnp.int32),
        entp, relp)
    return jnp.sum(partials) * jnp.float32(1.0 / B)


# speedup vs baseline: 1.8867x; 1.0513x over previous
import functools

import jax
import jax.numpy as jnp
from jax import lax
from jax.experimental import pallas as pl
from jax.experimental.pallas import tpu as pltpu
from jax.experimental.pallas import tpu_sc as plsc

_DIM = 64
_L = 16
_MARGIN = 1.0


_BLK = 32768
_SH = 15
_HW = _BLK // 2


def _pack_body(x, out):
    out[:, :_DIM] = x[:, :_HW].T
    out[:, _DIM:] = x[:, _HW:].T


def _pack_table(tab_t):
    n = tab_t.shape[1]
    grid = (n + _BLK - 1) // _BLK
    return pl.pallas_call(
        _pack_body,
        grid=(grid,),
        in_specs=[pl.BlockSpec((_DIM, _BLK), lambda j: (0, j))],
        out_specs=pl.BlockSpec((_HW, 2 * _DIM), lambda j: (j, 0)),
        out_shape=jax.ShapeDtypeStruct((grid * _HW, 2 * _DIM), jnp.float32),
    )(tab_t)


def _transe_body(C, n_chunks,
                 rph, rpr, rpt, rnh, rnr, rnt, entp, relp, out,
                 iph, ipr, ipt, inh, inr, int_,
                 sph, spr, spt, snh, snr, snt,
                 vph, vpr, vpt, vnh, vnr, vnt, sbuf, acc_ref, sem):
    nc = 2
    wid = lax.axis_index("s") * nc + lax.axis_index("c")
    base = wid * (C * n_chunks)
    lane = lax.iota(jnp.int32, _L)

    acc_ref[...] = jnp.zeros((_L,), jnp.float32)
    for c in range(n_chunks):
        off = base + c * C
        pltpu.sync_copy(rph.at[pl.ds(off, C)], iph)
        pltpu.sync_copy(rpr.at[pl.ds(off, C)], ipr)
        pltpu.sync_copy(rpt.at[pl.ds(off, C)], ipt)
        pltpu.sync_copy(rnh.at[pl.ds(off, C)], inh)
        pltpu.sync_copy(rnr.at[pl.ds(off, C)], inr)
        pltpu.sync_copy(rnt.at[pl.ds(off, C)], int_)
        for src, dst in ((iph, sph), (ipr, spr), (ipt, spt),
                         (inh, snh), (inr, snr), (int_, snt)):
            for s in range(C // _L):
                sl = pl.ds(s * _L, _L)
                iv = src[sl]
                dst[sl] = (lax.shift_right_logical(iv, _SH) * _HW
                           + (iv & (_HW - 1)))
        d0 = pltpu.async_copy(entp.at[sph], vph, sem)
        d1 = pltpu.async_copy(relp.at[spr], vpr, sem)
        d2 = pltpu.async_copy(entp.at[spt], vpt, sem)
        d3 = pltpu.async_copy(entp.at[snh], vnh, sem)
        d4 = pltpu.async_copy(relp.at[snr], vnr, sem)
        d5 = pltpu.async_copy(entp.at[snt], vnt, sem)
        d0.wait(); d1.wait(); d2.wait(); d3.wait(); d4.wait(); d5.wait()

        def group(g, carry):
            gs = pl.ds(g * _L, _L)
            hph, hpr, hpt = iph[gs] & _HW, ipr[gs] & _HW, ipt[gs] & _HW
            hnh, hnr, hnt = inh[gs] & _HW, inr[gs] & _HW, int_[gs] & _HW
            for j in range(_L):
                i = g * _L + j
                s = None
                for k in range(_DIM // _L):
                    sl = pl.ds(k * _L, _L)
                    sh = pl.ds(_DIM + k * _L, _L)
                    ph = jnp.where(hph[j] != 0, vph[i, sh], vph[i, sl])
                    pr = jnp.where(hpr[j] != 0, vpr[i, sh], vpr[i, sl])
                    pt = jnp.where(hpt[j] != 0, vpt[i, sh], vpt[i, sl])
                    nh = jnp.where(hnh[j] != 0, vnh[i, sh], vnh[i, sl])
                    nr = jnp.where(hnr[j] != 0, vnr[i, sh], vnr[i, sl])
                    nt = jnp.where(hnt[j] != 0, vnt[i, sh], vnt[i, sl])
                    d = jnp.abs(ph + pr - pt) - jnp.abs(nh + nr - nt)
                    s = d if s is None else s + d
                plsc.store_scatter(sbuf, [lane * _L + j], s)
            tot = None
            for k in range(_L):
                v = sbuf[pl.ds(k * _L, _L)]
                tot = v if tot is None else tot + v
            hinge = jnp.maximum(tot + _MARGIN, 0.0)
            acc_ref[...] = acc_ref[...] + hinge
            return carry

        lax.fori_loop(0, C // _L, group, jnp.int32(0))

    pltpu.sync_copy(acc_ref, out.at[pl.ds(wid * _L, _L)])


def _transe_sc(rph, rpr, rpt, rnh, rnr, rnt, entp, relp):
    B = rph.shape[0]
    nw = 32
    C = 128
    n_chunks = B // (nw * C)
    mesh = plsc.VectorSubcoreMesh(core_axis_name="c", subcore_axis_name="s")
    idx_t = pltpu.VMEM((C,), jnp.int32)
    row_t = pltpu.VMEM((C, 2 * _DIM), jnp.float32)
    kern = pl.kernel(
        functools.partial(_transe_body, C, n_chunks),
        mesh=mesh,
        compiler_params=pltpu.CompilerParams(needs_layout_passes=False),
        out_type=jax.ShapeDtypeStruct((nw * _L,), jnp.float32),
        scratch_types=[idx_t] * 12 + [row_t] * 6 + [
            pltpu.VMEM((_L * _L,), jnp.float32),
            pltpu.VMEM((_L,), jnp.float32),
            pltpu.SemaphoreType.DMA,
        ],
    )
    return kern(rph, rpr, rpt, rnh, rnr, rnt, entp, relp)


def kernel(r_p_h, r_p_r, r_p_t, r_n_h, r_n_r, r_n_t, ent_embed, rel_embed):
    B = r_p_h.shape[0]
    entp = _pack_table(ent_embed.T)
    relp = _pack_table(rel_embed.T)
    partials = _transe_sc(
        r_p_h.astype(jnp.int32), r_p_r.astype(jnp.int32),
        r_p_t.astype(jnp.int32), r_n_h.astype(jnp.int32),
        r_n_r.astype(jnp.int32), r_n_t.astype(jnp.int32),
        entp, relp)
    return jnp.sum(partials) * jnp.float32(1.0 / B)
